# baseline (device time: 151485 ns/iter reference)
import jax
import jax.numpy as jnp
from jax import lax
from jax.experimental import pallas as pl
from jax.experimental.pallas import tpu as pltpu

N_DEV = 8
ORDERS = ((1, 3, 4), (3, 4, 1), (4, 1, 3))
SPLITS = ((0, 1368), (1368, 1368), (2736, 1360))
HELD2 = ((0, 1, 3, 2), (0, 3, 4, 7), (0, 4, 1, 5))
RECV2 = tuple(
    tuple(j ^ ORDERS[r][2] for j in HELD2[r]) for r in range(3)
)


def kernel(x, w_mat):
    m_per, k = x.shape
    _, n_per = w_mat.shape
    m_total = N_DEV * m_per

    def body(x_ref, w_ref, out_ref, wg, res_s, res_r,
             wsend_sems, wrecv_sems, rsend_sems, rrecv_sems):
        my = lax.axis_index("i")

        barrier_sem = pltpu.get_barrier_semaphore()
        for m in (1, 3, 4):
            pl.semaphore_signal(
                barrier_sem, inc=1,
                device_id=(my ^ m,), device_id_type=pl.DeviceIdType.MESH,
            )
        pl.semaphore_wait(barrier_sem, 3)

        def w_slot(j, off, ln):
            if j == 0:
                return w_ref.at[pl.ds(off, ln), :]
            return wg.at[j - 1, pl.ds(off, ln), :]

        sem_i = 0

        def make_w(r, p, j):
            nonlocal sem_i
            m = ORDERS[r][p]
            off, ln = SPLITS[r]
            d = pltpu.make_async_remote_copy(
                src_ref=w_slot(j, off, ln),
                dst_ref=w_slot(j ^ m, off, ln),
                send_sem=wsend_sems.at[sem_i],
                recv_sem=wrecv_sems.at[sem_i],
                device_id=(my ^ m,),
                device_id_type=pl.DeviceIdType.MESH,
            )
            sem_i += 1
            return d

        d0 = [make_w(r, 0, 0) for r in range(3)]
        d1 = [[make_w(r, 1, j) for j in (0, ORDERS[r][0])] for r in range(3)]
        d2 = [[make_w(r, 2, j) for j in HELD2[r]] for r in range(3)]

        d_res = {}
        for delta in range(1, N_DEV):
            d_res[delta] = pltpu.make_async_remote_copy(
                src_ref=res_s.at[delta - 1],
                dst_ref=res_r.at[delta - 1],
                send_sem=rsend_sems.at[delta - 1],
                recv_sem=rrecv_sems.at[delta - 1],
                device_id=(my ^ delta,),
                device_id_type=pl.DeviceIdType.MESH,
            )

        def accum(r, j):
            off, ln = SPLITS[r]
            res_s[j - 1] = res_s[j - 1] + jnp.dot(
                x_ref[:, pl.ds(off, ln)], wg[j - 1, pl.ds(off, ln), :],
                preferred_element_type=jnp.float32,
            )

        for r in range(3):
            d0[r].start()
        for r in range(3):
            d1[r][0].start()
        for r in range(3):
            d2[r][0].start()
        res_s[...] = jnp.zeros((N_DEV - 1, m_per, n_per), jnp.float32)
        out_ref[pl.ds(my * m_per, m_per), :] = jnp.dot(
            x_ref[...], w_ref[...], preferred_element_type=jnp.float32,
        )

        for r in range(3):
            d0[r].wait_recv()
            d1[r][1].start()
            d2[r][1].start()
        for r in range(3):
            accum(r, ORDERS[r][0])

        for r in range(3):
            d1[r][0].wait_recv()
            d1[r][1].wait_recv()
            d2[r][2].start()
            d2[r][3].start()
        for r in range(3):
            m1, m2 = ORDERS[r][0], ORDERS[r][1]
            accum(r, m2)
            accum(r, m1 ^ m2)

        complete_after = {
            (0, 0): 4, (1, 0): 1, (2, 0): 3,
            (0, 2): 7, (1, 2): 5, (2, 2): 2,
            (2, 3): 6,
        }
        for i in range(4):
            for r in range(3):
                d2[r][i].wait_recv()
                accum(r, RECV2[r][i])
                delta = complete_after.get((r, i))
                if delta is not None:
                    d_res[delta].start()

        for delta in (4, 1, 3, 7, 5, 2, 6):
            d_res[delta].wait_recv()
            origin = my ^ delta
            out_ref[pl.ds(origin * m_per, m_per), :] = res_r[delta - 1]

        for r in range(3):
            d0[r].wait_send()
            for d in d1[r]:
                d.wait_send()
            for d in d2[r]:
                d.wait_send()
        for delta in range(1, N_DEV):
            d_res[delta].wait_send()

    n_w = 21
    return pl.pallas_call(
        body,
        out_shape=jax.ShapeDtypeStruct((m_total, n_per), jnp.float32),
        in_specs=[
            pl.BlockSpec(memory_space=pltpu.VMEM),
            pl.BlockSpec(memory_space=pltpu.VMEM),
        ],
        out_specs=pl.BlockSpec(memory_space=pltpu.VMEM),
        scratch_shapes=[
            pltpu.VMEM((N_DEV - 1, k, n_per), x.dtype),
            pltpu.VMEM((N_DEV - 1, m_per, n_per), x.dtype),
            pltpu.VMEM((N_DEV - 1, m_per, n_per), x.dtype),
            pltpu.SemaphoreType.DMA((n_w,)),
            pltpu.SemaphoreType.DMA((n_w,)),
            pltpu.SemaphoreType.DMA((N_DEV - 1,)),
            pltpu.SemaphoreType.DMA((N_DEV - 1,)),
        ],
        compiler_params=pltpu.CompilerParams(
            collective_id=0,
            vmem_limit_bytes=100 * 1024 * 1024,
        ),
    )(x, w_mat)
